# baseline (speedup 1.0000x reference)
import jax
import jax.numpy as jnp
from jax.experimental import pallas as pl
from jax.experimental.pallas import tpu as pltpu

B, T, D = 4096, 25, 512
CB = 128
NCHUNK = B // CB
NBUF = 5
OBUF = 5


def _prep_kernel(mi_ref, ip_ref, vmt_ref, mte_ref, me_ref, w_ref, b_ref,
                 extra_ref, wp_ref):
    m = mi_ref[0]
    pf = jnp.where(ip_ref[0] != 0, 1.0, 0.0).astype(jnp.float32)

    oh_type = (jax.lax.broadcasted_iota(jnp.int32, (1, 5), 1) == m
               ).astype(jnp.float32)
    type_emb = jnp.dot(oh_type, mte_ref[...],
                       preferred_element_type=jnp.float32)
    mask_idx = jnp.where(ip_ref[0] != 0, 1, 0)
    oh_mask = (jax.lax.broadcasted_iota(jnp.int32, (1, 2), 1) == mask_idx
               ).astype(jnp.float32)
    mask_emb = jnp.dot(oh_mask, me_ref[...],
                       preferred_element_type=jnp.float32)
    miss = vmt_ref[m]
    extra_x = (1.0 - pf) * miss + (type_emb + mask_emb)
    extra_ref[...] = jax.lax.dot_general(
        extra_x, w_ref[...], (((1,), (1,)), ((), ())),
        preferred_element_type=jnp.float32) + b_ref[...]
    wp_ref[...] = (pf * w_ref[...]).astype(jnp.bfloat16)


def _stream_kernel(tok_hbm, wp_ref, extra_ref, out_hbm,
                   ibuf, obuf, isem, osem):
    i = pl.program_id(0)

    def in_copy(chunk, slot):
        return pltpu.make_async_copy(
            tok_hbm.at[pl.ds(chunk * CB, CB)], ibuf.at[slot], isem.at[slot])

    def out_copy(chunk, slot):
        return pltpu.make_async_copy(
            obuf.at[slot], out_hbm.at[pl.ds(chunk * CB, CB)], osem.at[slot])

    @pl.when(i == 0)
    def _prologue():
        for k in range(NBUF - 1):
            in_copy(k, k).start()

    nxt = i + NBUF - 1

    @pl.when(nxt < NCHUNK)
    def _issue_ahead():
        in_copy(nxt, jax.lax.rem(nxt, NBUF)).start()

    islot = jax.lax.rem(i, NBUF)
    oslot = jax.lax.rem(i, OBUF)
    in_copy(i, islot).wait()

    @pl.when(i >= OBUF)
    def _reclaim():
        out_copy(i - OBUF, oslot).wait()

    x = ibuf[islot]
    y = jax.lax.dot_general(
        x, wp_ref[...], (((2,), (1,)), ((), ())),
        preferred_element_type=jnp.float32) + extra_ref[...][None, :, :]
    obuf[oslot] = y.astype(jnp.bfloat16)
    out_copy(i, oslot).start()

    @pl.when(i == NCHUNK - 1)
    def _drain():
        for k in range(OBUF):
            c = NCHUNK - OBUF + k
            out_copy(c, c % OBUF).wait()


@jax.jit
def kernel(tokens, modality_type_embeddings, vision_missing_tokens,
           text_missing_tokens, mask_embeddings, W, b,
           modality_idx, is_present):
    del text_missing_tokens
    mi = jnp.asarray(modality_idx, jnp.int32).reshape(1)
    ip = jnp.asarray(is_present, jnp.int32).reshape(1)
    b2 = b.reshape(1, D)

    extra, wp = pl.pallas_call(
        _prep_kernel,
        in_specs=[
            pl.BlockSpec(memory_space=pltpu.SMEM),
            pl.BlockSpec(memory_space=pltpu.SMEM),
            pl.BlockSpec((4, T, D), lambda: (0, 0, 0)),
            pl.BlockSpec((5, D), lambda: (0, 0)),
            pl.BlockSpec((2, D), lambda: (0, 0)),
            pl.BlockSpec((D, D), lambda: (0, 0)),
            pl.BlockSpec((1, D), lambda: (0, 0)),
        ],
        out_specs=[
            pl.BlockSpec((T, D), lambda: (0, 0)),
            pl.BlockSpec((D, D), lambda: (0, 0)),
        ],
        out_shape=[
            jax.ShapeDtypeStruct((T, D), jnp.float32),
            jax.ShapeDtypeStruct((D, D), jnp.bfloat16),
        ],
    )(mi, ip, vision_missing_tokens, modality_type_embeddings,
      mask_embeddings, W, b2)

    out16 = pl.pallas_call(
        _stream_kernel,
        grid=(NCHUNK,),
        in_specs=[
            pl.BlockSpec(memory_space=pl.ANY),
            pl.BlockSpec((D, D), lambda i: (0, 0)),
            pl.BlockSpec((T, D), lambda i: (0, 0)),
        ],
        out_specs=pl.BlockSpec(memory_space=pl.ANY),
        out_shape=jax.ShapeDtypeStruct((B, T, D), jnp.bfloat16),
        scratch_shapes=[
            pltpu.VMEM((NBUF, CB, T, D), jnp.bfloat16),
            pltpu.VMEM((OBUF, CB, T, D), jnp.bfloat16),
            pltpu.SemaphoreType.DMA((NBUF,)),
            pltpu.SemaphoreType.DMA((OBUF,)),
        ],
        compiler_params=pltpu.CompilerParams(
            dimension_semantics=("arbitrary",)),
    )(tokens.astype(jnp.bfloat16), wp, extra)
    return out16.astype(jnp.float32)
